# baseline (speedup 1.0000x reference)
import functools

import jax
import jax.numpy as jnp
from jax import lax
from jax.experimental import pallas as pl
from jax.experimental.pallas import tpu as pltpu
from jax.experimental.pallas import tpu_sc as plsc

NC = 2
NS = 16
NW = NC * NS
LANES = 16
BATCH = 16384
EMBED = 64
VOCAB = 100000
BPW = BATCH // NW
CHUNK = 128
NCH = BPW // CHUNK
PAIR = 1024
ZW = 2 * EMBED
NBLK = 49
ZROWS = NBLK * PAIR

_MESH = dict(core_axis_name="c", subcore_axis_name="s",
             num_cores=NC, num_subcores=NS)
_PARAMS = pltpu.CompilerParams(
    use_tc_tiling_on_sc=False, needs_layout_passes=False)


def _tc_repack(et):
    def body(a_ref, o_ref):
        a = a_ref[...]
        o_ref[...] = jnp.concatenate(
            [a[:, :PAIR].T, a[:, PAIR:].T], axis=1)

    return pl.pallas_call(
        body,
        grid=(NBLK,),
        in_specs=[pl.BlockSpec((EMBED, 2 * PAIR), lambda j: (0, j))],
        out_specs=pl.BlockSpec((PAIR, ZW), lambda j: (j, 0)),
        out_shape=jax.ShapeDtypeStruct((ZROWS, ZW), jnp.float32),
    )(et)


def _sc_gather_u(idxcols, zu, user_bias_flat):
    @functools.partial(
        pl.kernel,
        out_type=(
            jax.ShapeDtypeStruct((NW, BPW, ZW), jnp.float32),
            jax.ShapeDtypeStruct((NW, NCH, CHUNK), jnp.float32),
        ),
        mesh=plsc.VectorSubcoreMesh(**_MESH),
        compiler_params=_PARAMS,
        scratch_types=[
            pltpu.VMEM((NCH, CHUNK), jnp.int32),
            pltpu.VMEM((NCH, CHUNK), jnp.int32),
            pltpu.VMEM((BPW, ZW), jnp.float32),
            pltpu.VMEM((NCH, CHUNK), jnp.float32),
            pltpu.SemaphoreType.DMA,
            pltpu.SemaphoreType.DMA,
        ],
    )
    def ka(idx_h, zu_h, ubias_h, ug_h, ubg_h,
           idxu_v, zru_v, urows_v, ub_v, sem_u, sem_b):
        wid = lax.axis_index("s") * NC + lax.axis_index("c")
        pltpu.sync_copy(idx_h.at[0, wid], idxu_v)
        for j in range(NCH):
            for k in range(CHUNK // LANES):
                sl = pl.ds(k * LANES, LANES)
                ru = idxu_v[j, sl]
                zru_v[j, sl] = ((ru >> 11) << 10) | (ru & (PAIR - 1))
        copies = []
        for j in range(NCH):
            copies.append(pltpu.async_copy(
                zu_h.at[zru_v.at[j]], urows_v.at[pl.ds(j * CHUNK, CHUNK)],
                sem_u))
            copies.append(pltpu.async_copy(
                ubias_h.at[idxu_v.at[j]], ub_v.at[j], sem_b))
        for c in copies:
            c.wait()
        pltpu.sync_copy(urows_v, ug_h.at[wid])
        pltpu.sync_copy(ub_v, ubg_h.at[wid])

    return ka(idxcols, zu, user_bias_flat)


def _sc_dot_v(idxcols, zv, item_bias_flat, ug):
    @functools.partial(
        pl.kernel,
        out_type=(
            jax.ShapeDtypeStruct((NW, LANES), jnp.float32),
            jax.ShapeDtypeStruct((NW, NCH, CHUNK), jnp.float32),
        ),
        mesh=plsc.VectorSubcoreMesh(**_MESH),
        compiler_params=_PARAMS,
        scratch_types=[
            pltpu.VMEM((NCH, CHUNK), jnp.int32),
            pltpu.VMEM((NCH, CHUNK), jnp.int32),
            pltpu.VMEM((NCH, CHUNK), jnp.int32),
            pltpu.VMEM((2, CHUNK, ZW), jnp.float32),
            pltpu.VMEM((2, CHUNK, ZW), jnp.float32),
            pltpu.VMEM((NCH, CHUNK), jnp.float32),
            pltpu.VMEM((LANES,), jnp.float32),
            pltpu.SemaphoreType.DMA,
            pltpu.SemaphoreType.DMA,
            pltpu.SemaphoreType.DMA,
        ],
    )
    def kb(idx_h, zv_h, ibias_h, ug_h, parts_h, ibg_h,
           idxu_v, idxi_v, zri_v, urows_v, vrows_v, ib_v, acc_v,
           sem_u, sem_v, sem_b):
        wid = lax.axis_index("s") * NC + lax.axis_index("c")
        pltpu.sync_copy(idx_h.at[0, wid], idxu_v)
        pltpu.sync_copy(idx_h.at[1, wid], idxi_v)
        for j in range(NCH):
            for k in range(CHUNK // LANES):
                sl = pl.ds(k * LANES, LANES)
                ri = idxi_v[j, sl]
                zri_v[j, sl] = ((ri >> 11) << 10) | (ri & (PAIR - 1))
        bias_copies = []
        for j in range(NCH):
            bias_copies.append(pltpu.async_copy(
                ibias_h.at[idxi_v.at[j]], ib_v.at[j], sem_b))

        def fire(j):
            cu = pltpu.async_copy(
                ug_h.at[wid, pl.ds(j * CHUNK, CHUNK)], urows_v.at[j % 2],
                sem_u)
            cv = pltpu.async_copy(zv_h.at[zri_v.at[j]], vrows_v.at[j % 2],
                                  sem_v)
            return cu, cv

        inflight = fire(0)
        acc = jnp.zeros((LANES,), jnp.float32)
        for j in range(NCH):
            cu, cv = inflight
            if j + 1 < NCH:
                nxt = fire(j + 1)
            cu.wait()
            cv.wait()
            if j + 1 < NCH:
                inflight = nxt
            ub = urows_v.at[j % 2]
            vb = vrows_v.at[j % 2]

            def gbody(g, a, j=j, ub=ub, vb=vb):
                sl = pl.ds(g * LANES, LANES)
                offu16 = ((idxu_v[j, sl] >> 10) & 1) << 6
                offi16 = ((idxi_v[j, sl] >> 10) & 1) << 6
                base = g * LANES
                for ln in range(LANES):
                    su = offu16[ln]
                    si = offi16[ln]
                    row = base + ln
                    p = (ub[row, pl.ds(su, LANES)]
                         * vb[row, pl.ds(si, LANES)])
                    for c in range(1, EMBED // LANES):
                        p = p + (ub[row, pl.ds(su + c * LANES, LANES)]
                                 * vb[row, pl.ds(si + c * LANES, LANES)])
                    a = a + p
                return a

            acc = lax.fori_loop(0, CHUNK // LANES, gbody, acc)
        for c in bias_copies:
            c.wait()
        pltpu.sync_copy(ib_v, ibg_h.at[wid])
        acc_v[...] = acc
        pltpu.sync_copy(acc_v, parts_h.at[wid])

    return kb(idxcols, zv, item_bias_flat, ug)


def _sc_finish(parts, ubg, ibg):
    @functools.partial(
        pl.kernel,
        out_type=jax.ShapeDtypeStruct((NW, BPW), jnp.float32),
        mesh=plsc.VectorSubcoreMesh(**_MESH),
        compiler_params=_PARAMS,
        scratch_types=[
            pltpu.VMEM((NW, LANES), jnp.float32),
            pltpu.VMEM((BPW,), jnp.float32),
            pltpu.VMEM((BPW,), jnp.float32),
            pltpu.VMEM((BPW,), jnp.float32),
        ],
    )
    def fin_kernel(parts_h, ub_h, ib_h, out_h, parts_v, ub_v, ib_v, out_v):
        wid = lax.axis_index("s") * NC + lax.axis_index("c")
        pltpu.sync_copy(parts_h, parts_v)
        pltpu.sync_copy(ub_h.at[wid], ub_v)
        pltpu.sync_copy(ib_h.at[wid], ib_v)
        acc = jnp.zeros((LANES,), jnp.float32)
        for w in range(NW):
            acc = acc + parts_v[w, :]
        s = jnp.sum(acc)
        for g in range(BPW // LANES):
            sl = pl.ds(g * LANES, LANES)
            x = s + ub_v[sl] + ib_v[sl]
            out_v[sl] = 1.0 / (1.0 + jnp.exp(-x))
        pltpu.sync_copy(out_v, out_h.at[wid])

    return fin_kernel(parts, ubg, ibg)


def kernel(inputs, user_embedding, user_bias, item_embedding, item_bias):
    idxcols = inputs.T.reshape(2, NW, NCH, CHUNK)
    zu = _tc_repack(user_embedding.T)
    ug, ubg = _sc_gather_u(idxcols, zu, user_bias.reshape(-1))
    zv = _tc_repack(item_embedding.T)
    parts, ibg = _sc_dot_v(idxcols, zv, item_bias.reshape(-1), ug)
    out = _sc_finish(parts, ubg.reshape(NW, BPW), ibg.reshape(NW, BPW))
    return out.reshape(BATCH, 1)


# speedup vs baseline: 1.0443x; 1.0443x over previous
import functools

import jax
import jax.numpy as jnp
from jax import lax
from jax.experimental import pallas as pl
from jax.experimental.pallas import tpu as pltpu
from jax.experimental.pallas import tpu_sc as plsc

NC = 2
NS = 16
NW = NC * NS
LANES = 16
BATCH = 16384
EMBED = 64
VOCAB = 100000
BPW = BATCH // NW
CHUNK = 128
NCH = BPW // CHUNK
PAIR = 1024
ZW = 2 * EMBED
NBLK = 49
ZROWS = NBLK * PAIR

_MESH = dict(core_axis_name="c", subcore_axis_name="s",
             num_cores=NC, num_subcores=NS)
_PARAMS = pltpu.CompilerParams(
    use_tc_tiling_on_sc=False, needs_layout_passes=False)


def _tc_repack(et):
    def body(a_ref, o_ref):
        a = a_ref[...]
        o_ref[...] = jnp.concatenate(
            [a[:, :PAIR].T, a[:, PAIR:].T], axis=1)

    return pl.pallas_call(
        body,
        grid=(NBLK,),
        in_specs=[pl.BlockSpec((EMBED, 2 * PAIR), lambda j: (0, j))],
        out_specs=pl.BlockSpec((PAIR, ZW), lambda j: (j, 0)),
        out_shape=jax.ShapeDtypeStruct((ZROWS, ZW), jnp.float32),
    )(et)


def _sc_gather_u(idxcols, zu, user_bias_flat):
    @functools.partial(
        pl.kernel,
        out_type=(
            jax.ShapeDtypeStruct((NW, BPW, ZW), jnp.float32),
            jax.ShapeDtypeStruct((NW, NCH, CHUNK), jnp.float32),
        ),
        mesh=plsc.VectorSubcoreMesh(**_MESH),
        compiler_params=_PARAMS,
        scratch_types=[
            pltpu.VMEM((NCH, CHUNK), jnp.int32),
            pltpu.VMEM((NCH, CHUNK), jnp.int32),
            pltpu.VMEM((BPW, ZW), jnp.float32),
            pltpu.VMEM((NCH, CHUNK), jnp.float32),
            pltpu.SemaphoreType.DMA,
            pltpu.SemaphoreType.DMA,
        ],
    )
    def ka(idx_h, zu_h, ubias_h, ug_h, ubg_h,
           idxu_v, zru_v, urows_v, ub_v, sem_u, sem_b):
        wid = lax.axis_index("s") * NC + lax.axis_index("c")
        pltpu.sync_copy(idx_h.at[0, wid], idxu_v)
        for j in range(NCH):
            for k in range(CHUNK // LANES):
                sl = pl.ds(k * LANES, LANES)
                ru = idxu_v[j, sl]
                zru_v[j, sl] = ((ru >> 11) << 10) | (ru & (PAIR - 1))
        copies = []
        for j in range(NCH):
            copies.append(pltpu.async_copy(
                zu_h.at[zru_v.at[j]], urows_v.at[pl.ds(j * CHUNK, CHUNK)],
                sem_u))
            copies.append(pltpu.async_copy(
                ubias_h.at[idxu_v.at[j]], ub_v.at[j], sem_b))
        for c in copies:
            c.wait()
        pltpu.sync_copy(urows_v, ug_h.at[wid])
        pltpu.sync_copy(ub_v, ubg_h.at[wid])

    return ka(idxcols, zu, user_bias_flat)


def _sc_dot_v(idxcols, zv, item_bias_flat, ug):
    @functools.partial(
        pl.kernel,
        out_type=(
            jax.ShapeDtypeStruct((NW, LANES), jnp.float32),
            jax.ShapeDtypeStruct((NW, NCH, CHUNK), jnp.float32),
        ),
        mesh=plsc.VectorSubcoreMesh(**_MESH),
        compiler_params=_PARAMS,
        scratch_types=[
            pltpu.VMEM((NCH, CHUNK), jnp.int32),
            pltpu.VMEM((NCH, CHUNK), jnp.int32),
            pltpu.VMEM((NCH, CHUNK), jnp.int32),
            pltpu.VMEM((2, CHUNK, ZW), jnp.float32),
            pltpu.VMEM((2, CHUNK, ZW), jnp.float32),
            pltpu.VMEM((NCH, CHUNK), jnp.float32),
            pltpu.VMEM((LANES,), jnp.float32),
            pltpu.SemaphoreType.DMA,
            pltpu.SemaphoreType.DMA,
            pltpu.SemaphoreType.DMA,
        ],
    )
    def kb(idx_h, zv_h, ibias_h, ug_h, parts_h, ibg_h,
           idxu_v, idxi_v, zri_v, urows_v, vrows_v, ib_v, acc_v,
           sem_u, sem_v, sem_b):
        wid = lax.axis_index("s") * NC + lax.axis_index("c")
        pltpu.sync_copy(idx_h.at[0, wid], idxu_v)
        pltpu.sync_copy(idx_h.at[1, wid], idxi_v)
        for j in range(NCH):
            for k in range(CHUNK // LANES):
                sl = pl.ds(k * LANES, LANES)
                ri = idxi_v[j, sl]
                zri_v[j, sl] = ((ri >> 11) << 10) | (ri & (PAIR - 1))
        bias_copies = []
        for j in range(NCH):
            bias_copies.append(pltpu.async_copy(
                ibias_h.at[idxi_v.at[j]], ib_v.at[j], sem_b))

        def fire(j):
            cu = pltpu.async_copy(
                ug_h.at[wid, pl.ds(j * CHUNK, CHUNK)], urows_v.at[j % 2],
                sem_u)
            cv = pltpu.async_copy(zv_h.at[zri_v.at[j]], vrows_v.at[j % 2],
                                  sem_v)
            return cu, cv

        inflight = fire(0)
        acc = jnp.zeros((LANES,), jnp.float32)
        for j in range(NCH):
            cu, cv = inflight
            if j + 1 < NCH:
                nxt = fire(j + 1)
            cu.wait()
            cv.wait()
            if j + 1 < NCH:
                inflight = nxt
            ub = urows_v.at[j % 2]
            vb = vrows_v.at[j % 2]

            def gbody(g, a, j=j, ub=ub, vb=vb):
                sl = pl.ds(g * LANES, LANES)
                offu16 = ((idxu_v[j, sl] >> 10) & 1) << 6
                offi16 = ((idxi_v[j, sl] >> 10) & 1) << 6
                base = g * LANES
                for ln in range(LANES):
                    su = offu16[ln]
                    si = offi16[ln]
                    row = base + ln
                    p = (ub[row, pl.ds(su, LANES)]
                         * vb[row, pl.ds(si, LANES)])
                    for c in range(1, EMBED // LANES):
                        p = p + (ub[row, pl.ds(su + c * LANES, LANES)]
                                 * vb[row, pl.ds(si + c * LANES, LANES)])
                    a = a + p
                return a

            acc = lax.fori_loop(0, CHUNK // LANES, gbody, acc)
        for c in bias_copies:
            c.wait()
        pltpu.sync_copy(ib_v, ibg_h.at[wid])
        acc_v[...] = acc
        pltpu.sync_copy(acc_v, parts_h.at[wid])

    return kb(idxcols, zv, item_bias_flat, ug)


def _sc_finish(parts, ubg, ibg):
    @functools.partial(
        pl.kernel,
        out_type=jax.ShapeDtypeStruct((NW, BPW), jnp.float32),
        mesh=plsc.VectorSubcoreMesh(**_MESH),
        compiler_params=_PARAMS,
        scratch_types=[
            pltpu.VMEM((NW, LANES), jnp.float32),
            pltpu.VMEM((BPW,), jnp.float32),
            pltpu.VMEM((BPW,), jnp.float32),
            pltpu.VMEM((BPW,), jnp.float32),
        ],
    )
    def fin_kernel(parts_h, ub_h, ib_h, out_h, parts_v, ub_v, ib_v, out_v):
        wid = lax.axis_index("s") * NC + lax.axis_index("c")
        pltpu.sync_copy(parts_h, parts_v)
        pltpu.sync_copy(ub_h.at[wid], ub_v)
        pltpu.sync_copy(ib_h.at[wid], ib_v)
        acc = jnp.zeros((LANES,), jnp.float32)
        for w in range(NW):
            acc = acc + parts_v[w, :]
        s = jnp.sum(acc)
        for g in range(BPW // LANES):
            sl = pl.ds(g * LANES, LANES)
            x = s + ub_v[sl] + ib_v[sl]
            out_v[sl] = 1.0 / (1.0 + jnp.exp(-x))
        pltpu.sync_copy(out_v, out_h.at[wid])

    return fin_kernel(parts, ubg, ibg)


def kernel(inputs, user_embedding, user_bias, item_embedding, item_bias):
    idxcols = inputs.T.reshape(2, NW, NCH, CHUNK)
    zu = _tc_repack(user_embedding.T)
    ug, ubg = _sc_gather_u(idxcols, zu, user_bias.reshape(-1))
    zv = _tc_repack(item_embedding.T)
    parts, ibg = _sc_dot_v(idxcols, zv, item_bias.reshape(-1), ug)
    out = _tc_finish(parts.reshape(NW * LANES // CHUNK, CHUNK),
                     ubg.reshape(CHUNK, CHUNK), ibg.reshape(CHUNK, CHUNK))
    return out.reshape(BATCH, 1)


def _tc_finish(parts2d, ub2d, ib2d):
    def tc_body(parts_ref, ub_ref, ib_ref, out_ref):
        s = jnp.sum(parts_ref[...])
        x = ub_ref[...] + ib_ref[...] + s
        out_ref[...] = 1.0 / (1.0 + jnp.exp(-x))

    return pl.pallas_call(
        tc_body,
        out_shape=jax.ShapeDtypeStruct(ub2d.shape, jnp.float32),
    )(parts2d, ub2d, ib2d)


# speedup vs baseline: 1.2637x; 1.2101x over previous
import functools

import jax
import jax.numpy as jnp
from jax import lax
from jax.experimental import pallas as pl
from jax.experimental.pallas import tpu as pltpu
from jax.experimental.pallas import tpu_sc as plsc

NC = 2
NS = 16
NW = NC * NS
LANES = 16
BATCH = 16384
EMBED = 64
VOCAB = 100000
BPW = BATCH // NW
CHUNK = 128
NCH = BPW // CHUNK
PAIR = 2048
ZW = 2 * EMBED
NBLK = 25
ZROWS = NBLK * PAIR

_MESH = dict(core_axis_name="c", subcore_axis_name="s",
             num_cores=NC, num_subcores=NS)
_PARAMS = pltpu.CompilerParams(
    use_tc_tiling_on_sc=False, needs_layout_passes=False)


def _tc_repack(et):
    def body(a_ref, o_ref):
        a = a_ref[...]
        o_ref[...] = jnp.concatenate(
            [a[:, :PAIR].T, a[:, PAIR:].T], axis=1)

    return pl.pallas_call(
        body,
        grid=(NBLK,),
        in_specs=[pl.BlockSpec((EMBED, 2 * PAIR), lambda j: (0, j))],
        out_specs=pl.BlockSpec((PAIR, ZW), lambda j: (j, 0)),
        out_shape=jax.ShapeDtypeStruct((ZROWS, ZW), jnp.float32),
    )(et)


def _sc_gather_u(idxcols, zu, user_bias_flat):
    @functools.partial(
        pl.kernel,
        out_type=(
            jax.ShapeDtypeStruct((NW, BPW, ZW), jnp.float32),
            jax.ShapeDtypeStruct((NW, NCH, CHUNK), jnp.float32),
        ),
        mesh=plsc.VectorSubcoreMesh(**_MESH),
        compiler_params=_PARAMS,
        scratch_types=[
            pltpu.VMEM((NCH, CHUNK), jnp.int32),
            pltpu.VMEM((NCH, CHUNK), jnp.int32),
            pltpu.VMEM((BPW, ZW), jnp.float32),
            pltpu.VMEM((NCH, CHUNK), jnp.float32),
            pltpu.SemaphoreType.DMA,
            pltpu.SemaphoreType.DMA,
        ],
    )
    def ka(idx_h, zu_h, ubias_h, ug_h, ubg_h,
           idxu_v, zru_v, urows_v, ub_v, sem_u, sem_b):
        wid = lax.axis_index("s") * NC + lax.axis_index("c")
        pltpu.sync_copy(idx_h.at[0, wid], idxu_v)
        for j in range(NCH):
            for k in range(CHUNK // LANES):
                sl = pl.ds(k * LANES, LANES)
                ru = idxu_v[j, sl]
                zru_v[j, sl] = ((ru >> 12) << 11) | (ru & (PAIR - 1))
        copies = []
        for j in range(NCH):
            copies.append(pltpu.async_copy(
                zu_h.at[zru_v.at[j]], urows_v.at[pl.ds(j * CHUNK, CHUNK)],
                sem_u))
            copies.append(pltpu.async_copy(
                ubias_h.at[idxu_v.at[j]], ub_v.at[j], sem_b))
        for c in copies:
            c.wait()
        pltpu.sync_copy(urows_v, ug_h.at[wid])
        pltpu.sync_copy(ub_v, ubg_h.at[wid])

    return ka(idxcols, zu, user_bias_flat)


def _sc_dot_v(idxcols, zv, item_bias_flat, ug):
    @functools.partial(
        pl.kernel,
        out_type=(
            jax.ShapeDtypeStruct((NW, LANES), jnp.float32),
            jax.ShapeDtypeStruct((NW, NCH, CHUNK), jnp.float32),
        ),
        mesh=plsc.VectorSubcoreMesh(**_MESH),
        compiler_params=_PARAMS,
        scratch_types=[
            pltpu.VMEM((NCH, CHUNK), jnp.int32),
            pltpu.VMEM((NCH, CHUNK), jnp.int32),
            pltpu.VMEM((NCH, CHUNK), jnp.int32),
            pltpu.VMEM((2, CHUNK, ZW), jnp.float32),
            pltpu.VMEM((2, CHUNK, ZW), jnp.float32),
            pltpu.VMEM((NCH, CHUNK), jnp.float32),
            pltpu.VMEM((LANES,), jnp.float32),
            pltpu.SemaphoreType.DMA,
            pltpu.SemaphoreType.DMA,
            pltpu.SemaphoreType.DMA,
        ],
    )
    def kb(idx_h, zv_h, ibias_h, ug_h, parts_h, ibg_h,
           idxu_v, idxi_v, zri_v, urows_v, vrows_v, ib_v, acc_v,
           sem_u, sem_v, sem_b):
        wid = lax.axis_index("s") * NC + lax.axis_index("c")
        pltpu.sync_copy(idx_h.at[0, wid], idxu_v)
        pltpu.sync_copy(idx_h.at[1, wid], idxi_v)
        for j in range(NCH):
            for k in range(CHUNK // LANES):
                sl = pl.ds(k * LANES, LANES)
                ri = idxi_v[j, sl]
                zri_v[j, sl] = ((ri >> 12) << 11) | (ri & (PAIR - 1))
        bias_copies = []
        for j in range(NCH):
            bias_copies.append(pltpu.async_copy(
                ibias_h.at[idxi_v.at[j]], ib_v.at[j], sem_b))

        def fire(j):
            cu = pltpu.async_copy(
                ug_h.at[wid, pl.ds(j * CHUNK, CHUNK)], urows_v.at[j % 2],
                sem_u)
            cv = pltpu.async_copy(zv_h.at[zri_v.at[j]], vrows_v.at[j % 2],
                                  sem_v)
            return cu, cv

        inflight = fire(0)
        acc = jnp.zeros((LANES,), jnp.float32)
        for j in range(NCH):
            cu, cv = inflight
            if j + 1 < NCH:
                nxt = fire(j + 1)
            cu.wait()
            cv.wait()
            if j + 1 < NCH:
                inflight = nxt
            ub = urows_v.at[j % 2]
            vb = vrows_v.at[j % 2]

            def gbody(g, a, j=j, ub=ub, vb=vb):
                sl = pl.ds(g * LANES, LANES)
                offu16 = ((idxu_v[j, sl] >> 11) & 1) << 6
                offi16 = ((idxi_v[j, sl] >> 11) & 1) << 6
                base = g * LANES
                for ln in range(LANES):
                    su = offu16[ln]
                    si = offi16[ln]
                    row = base + ln
                    p = (ub[row, pl.ds(su, LANES)]
                         * vb[row, pl.ds(si, LANES)])
                    for c in range(1, EMBED // LANES):
                        p = p + (ub[row, pl.ds(su + c * LANES, LANES)]
                                 * vb[row, pl.ds(si + c * LANES, LANES)])
                    a = a + p
                return a

            acc = lax.fori_loop(0, CHUNK // LANES, gbody, acc)
        for c in bias_copies:
            c.wait()
        pltpu.sync_copy(ib_v, ibg_h.at[wid])
        acc_v[...] = acc
        pltpu.sync_copy(acc_v, parts_h.at[wid])

    return kb(idxcols, zv, item_bias_flat, ug)


def _sc_finish(parts, ubg, ibg):
    @functools.partial(
        pl.kernel,
        out_type=jax.ShapeDtypeStruct((NW, BPW), jnp.float32),
        mesh=plsc.VectorSubcoreMesh(**_MESH),
        compiler_params=_PARAMS,
        scratch_types=[
            pltpu.VMEM((NW, LANES), jnp.float32),
            pltpu.VMEM((BPW,), jnp.float32),
            pltpu.VMEM((BPW,), jnp.float32),
            pltpu.VMEM((BPW,), jnp.float32),
        ],
    )
    def fin_kernel(parts_h, ub_h, ib_h, out_h, parts_v, ub_v, ib_v, out_v):
        wid = lax.axis_index("s") * NC + lax.axis_index("c")
        pltpu.sync_copy(parts_h, parts_v)
        pltpu.sync_copy(ub_h.at[wid], ub_v)
        pltpu.sync_copy(ib_h.at[wid], ib_v)
        acc = jnp.zeros((LANES,), jnp.float32)
        for w in range(NW):
            acc = acc + parts_v[w, :]
        s = jnp.sum(acc)
        for g in range(BPW // LANES):
            sl = pl.ds(g * LANES, LANES)
            x = s + ub_v[sl] + ib_v[sl]
            out_v[sl] = 1.0 / (1.0 + jnp.exp(-x))
        pltpu.sync_copy(out_v, out_h.at[wid])

    return fin_kernel(parts, ubg, ibg)


def kernel(inputs, user_embedding, user_bias, item_embedding, item_bias):
    idxcols = inputs.T.reshape(2, NW, NCH, CHUNK)
    zu = _tc_repack(user_embedding.T)
    ug, ubg = _sc_gather_u(idxcols, zu, user_bias.reshape(-1))
    zv = _tc_repack(item_embedding.T)
    parts, ibg = _sc_dot_v(idxcols, zv, item_bias.reshape(-1), ug)
    out = _tc_finish(parts.reshape(NW * LANES // CHUNK, CHUNK),
                     ubg.reshape(CHUNK, CHUNK), ibg.reshape(CHUNK, CHUNK))
    return out.reshape(BATCH, 1)


def _tc_finish(parts2d, ub2d, ib2d):
    def tc_body(parts_ref, ub_ref, ib_ref, out_ref):
        s = jnp.sum(parts_ref[...])
        x = ub_ref[...] + ib_ref[...] + s
        out_ref[...] = 1.0 / (1.0 + jnp.exp(-x))

    return pl.pallas_call(
        tc_body,
        out_shape=jax.ShapeDtypeStruct(ub2d.shape, jnp.float32),
    )(parts2d, ub2d, ib2d)


# speedup vs baseline: 1.4137x; 1.1186x over previous
import functools

import jax
import jax.numpy as jnp
from jax import lax
from jax.experimental import pallas as pl
from jax.experimental.pallas import tpu as pltpu
from jax.experimental.pallas import tpu_sc as plsc

NC = 2
NS = 16
NW = NC * NS
LANES = 16
BATCH = 16384
EMBED = 64
VOCAB = 100000
BPW = BATCH // NW
CHUNK = 128
NCH = BPW // CHUNK
PAIR = 4096
ZW = 2 * EMBED
NBLK = 13
ZROWS = NBLK * PAIR

_MESH = dict(core_axis_name="c", subcore_axis_name="s",
             num_cores=NC, num_subcores=NS)
_PARAMS = pltpu.CompilerParams(
    use_tc_tiling_on_sc=False, needs_layout_passes=False)


def _tc_repack(et):
    def body(a_ref, o_ref):
        a = a_ref[...]
        o_ref[...] = jnp.concatenate(
            [a[:, :PAIR].T, a[:, PAIR:].T], axis=1)

    return pl.pallas_call(
        body,
        grid=(NBLK,),
        in_specs=[pl.BlockSpec((EMBED, 2 * PAIR), lambda j: (0, j))],
        out_specs=pl.BlockSpec((PAIR, ZW), lambda j: (j, 0)),
        out_shape=jax.ShapeDtypeStruct((ZROWS, ZW), jnp.float32),
    )(et)


def _sc_gather_u(idxcols, zu, user_bias_flat):
    @functools.partial(
        pl.kernel,
        out_type=(
            jax.ShapeDtypeStruct((NW, BPW, ZW), jnp.float32),
            jax.ShapeDtypeStruct((NW, NCH, CHUNK), jnp.float32),
        ),
        mesh=plsc.VectorSubcoreMesh(**_MESH),
        compiler_params=_PARAMS,
        scratch_types=[
            pltpu.VMEM((NCH, CHUNK), jnp.int32),
            pltpu.VMEM((NCH, CHUNK), jnp.int32),
            pltpu.VMEM((BPW, ZW), jnp.float32),
            pltpu.VMEM((NCH, CHUNK), jnp.float32),
            pltpu.SemaphoreType.DMA,
            pltpu.SemaphoreType.DMA,
        ],
    )
    def ka(idx_h, zu_h, ubias_h, ug_h, ubg_h,
           idxu_v, zru_v, urows_v, ub_v, sem_u, sem_b):
        wid = lax.axis_index("s") * NC + lax.axis_index("c")
        pltpu.sync_copy(idx_h.at[0, wid], idxu_v)
        for j in range(NCH):
            for k in range(CHUNK // LANES):
                sl = pl.ds(k * LANES, LANES)
                ru = idxu_v[j, sl]
                zru_v[j, sl] = ((ru >> 13) << 12) | (ru & (PAIR - 1))
        copies = []
        for j in range(NCH):
            copies.append(pltpu.async_copy(
                zu_h.at[zru_v.at[j]], urows_v.at[pl.ds(j * CHUNK, CHUNK)],
                sem_u))
            copies.append(pltpu.async_copy(
                ubias_h.at[idxu_v.at[j]], ub_v.at[j], sem_b))
        for c in copies:
            c.wait()
        pltpu.sync_copy(urows_v, ug_h.at[wid])
        pltpu.sync_copy(ub_v, ubg_h.at[wid])

    return ka(idxcols, zu, user_bias_flat)


def _sc_dot_v(idxcols, zv, item_bias_flat, ug):
    @functools.partial(
        pl.kernel,
        out_type=(
            jax.ShapeDtypeStruct((NW, LANES), jnp.float32),
            jax.ShapeDtypeStruct((NW, NCH, CHUNK), jnp.float32),
        ),
        mesh=plsc.VectorSubcoreMesh(**_MESH),
        compiler_params=_PARAMS,
        scratch_types=[
            pltpu.VMEM((NCH, CHUNK), jnp.int32),
            pltpu.VMEM((NCH, CHUNK), jnp.int32),
            pltpu.VMEM((NCH, CHUNK), jnp.int32),
            pltpu.VMEM((2, CHUNK, ZW), jnp.float32),
            pltpu.VMEM((2, CHUNK, ZW), jnp.float32),
            pltpu.VMEM((NCH, CHUNK), jnp.float32),
            pltpu.VMEM((LANES,), jnp.float32),
            pltpu.SemaphoreType.DMA,
            pltpu.SemaphoreType.DMA,
            pltpu.SemaphoreType.DMA,
        ],
    )
    def kb(idx_h, zv_h, ibias_h, ug_h, parts_h, ibg_h,
           idxu_v, idxi_v, zri_v, urows_v, vrows_v, ib_v, acc_v,
           sem_u, sem_v, sem_b):
        wid = lax.axis_index("s") * NC + lax.axis_index("c")
        pltpu.sync_copy(idx_h.at[0, wid], idxu_v)
        pltpu.sync_copy(idx_h.at[1, wid], idxi_v)
        for j in range(NCH):
            for k in range(CHUNK // LANES):
                sl = pl.ds(k * LANES, LANES)
                ri = idxi_v[j, sl]
                zri_v[j, sl] = ((ri >> 13) << 12) | (ri & (PAIR - 1))
        bias_copies = []
        for j in range(NCH):
            bias_copies.append(pltpu.async_copy(
                ibias_h.at[idxi_v.at[j]], ib_v.at[j], sem_b))

        def fire(j):
            cu = pltpu.async_copy(
                ug_h.at[wid, pl.ds(j * CHUNK, CHUNK)], urows_v.at[j % 2],
                sem_u)
            cv = pltpu.async_copy(zv_h.at[zri_v.at[j]], vrows_v.at[j % 2],
                                  sem_v)
            return cu, cv

        inflight = fire(0)
        acc = jnp.zeros((LANES,), jnp.float32)
        for j in range(NCH):
            cu, cv = inflight
            if j + 1 < NCH:
                nxt = fire(j + 1)
            cu.wait()
            cv.wait()
            if j + 1 < NCH:
                inflight = nxt
            ub = urows_v.at[j % 2]
            vb = vrows_v.at[j % 2]

            def gbody(g, a, j=j, ub=ub, vb=vb):
                sl = pl.ds(g * LANES, LANES)
                offu16 = ((idxu_v[j, sl] >> 12) & 1) << 6
                offi16 = ((idxi_v[j, sl] >> 12) & 1) << 6
                base = g * LANES
                for ln in range(LANES):
                    su = offu16[ln]
                    si = offi16[ln]
                    row = base + ln
                    p = (ub[row, pl.ds(su, LANES)]
                         * vb[row, pl.ds(si, LANES)])
                    for c in range(1, EMBED // LANES):
                        p = p + (ub[row, pl.ds(su + c * LANES, LANES)]
                                 * vb[row, pl.ds(si + c * LANES, LANES)])
                    a = a + p
                return a

            acc = lax.fori_loop(0, CHUNK // LANES, gbody, acc)
        for c in bias_copies:
            c.wait()
        pltpu.sync_copy(ib_v, ibg_h.at[wid])
        acc_v[...] = acc
        pltpu.sync_copy(acc_v, parts_h.at[wid])

    return kb(idxcols, zv, item_bias_flat, ug)


def _sc_finish(parts, ubg, ibg):
    @functools.partial(
        pl.kernel,
        out_type=jax.ShapeDtypeStruct((NW, BPW), jnp.float32),
        mesh=plsc.VectorSubcoreMesh(**_MESH),
        compiler_params=_PARAMS,
        scratch_types=[
            pltpu.VMEM((NW, LANES), jnp.float32),
            pltpu.VMEM((BPW,), jnp.float32),
            pltpu.VMEM((BPW,), jnp.float32),
            pltpu.VMEM((BPW,), jnp.float32),
        ],
    )
    def fin_kernel(parts_h, ub_h, ib_h, out_h, parts_v, ub_v, ib_v, out_v):
        wid = lax.axis_index("s") * NC + lax.axis_index("c")
        pltpu.sync_copy(parts_h, parts_v)
        pltpu.sync_copy(ub_h.at[wid], ub_v)
        pltpu.sync_copy(ib_h.at[wid], ib_v)
        acc = jnp.zeros((LANES,), jnp.float32)
        for w in range(NW):
            acc = acc + parts_v[w, :]
        s = jnp.sum(acc)
        for g in range(BPW // LANES):
            sl = pl.ds(g * LANES, LANES)
            x = s + ub_v[sl] + ib_v[sl]
            out_v[sl] = 1.0 / (1.0 + jnp.exp(-x))
        pltpu.sync_copy(out_v, out_h.at[wid])

    return fin_kernel(parts, ubg, ibg)


def kernel(inputs, user_embedding, user_bias, item_embedding, item_bias):
    idxcols = inputs.T.reshape(2, NW, NCH, CHUNK)
    zu = _tc_repack(user_embedding.T)
    ug, ubg = _sc_gather_u(idxcols, zu, user_bias.reshape(-1))
    zv = _tc_repack(item_embedding.T)
    parts, ibg = _sc_dot_v(idxcols, zv, item_bias.reshape(-1), ug)
    out = _tc_finish(parts.reshape(NW * LANES // CHUNK, CHUNK),
                     ubg.reshape(CHUNK, CHUNK), ibg.reshape(CHUNK, CHUNK))
    return out.reshape(BATCH, 1)


def _tc_finish(parts2d, ub2d, ib2d):
    def tc_body(parts_ref, ub_ref, ib_ref, out_ref):
        s = jnp.sum(parts_ref[...])
        x = ub_ref[...] + ib_ref[...] + s
        out_ref[...] = 1.0 / (1.0 + jnp.exp(-x))

    return pl.pallas_call(
        tc_body,
        out_shape=jax.ShapeDtypeStruct(ub2d.shape, jnp.float32),
    )(parts2d, ub2d, ib2d)


# speedup vs baseline: 1.4455x; 1.0226x over previous
import functools

import jax
import jax.numpy as jnp
from jax import lax
from jax.experimental import pallas as pl
from jax.experimental.pallas import tpu as pltpu
from jax.experimental.pallas import tpu_sc as plsc

NC = 2
NS = 16
NW = NC * NS
LANES = 16
BATCH = 16384
EMBED = 64
VOCAB = 100000
BPW = BATCH // NW
CHUNK = 128
NCH = BPW // CHUNK
PAIR = 8192
ZW = 2 * EMBED
NBLK = 7
ZROWS = NBLK * PAIR

_MESH = dict(core_axis_name="c", subcore_axis_name="s",
             num_cores=NC, num_subcores=NS)
_PARAMS = pltpu.CompilerParams(
    use_tc_tiling_on_sc=False, needs_layout_passes=False)


def _tc_repack(et):
    def body(a_ref, o_ref):
        a = a_ref[...]
        o_ref[...] = jnp.concatenate(
            [a[:, :PAIR].T, a[:, PAIR:].T], axis=1)

    return pl.pallas_call(
        body,
        grid=(NBLK,),
        in_specs=[pl.BlockSpec((EMBED, 2 * PAIR), lambda j: (0, j))],
        out_specs=pl.BlockSpec((PAIR, ZW), lambda j: (j, 0)),
        out_shape=jax.ShapeDtypeStruct((ZROWS, ZW), jnp.float32),
    )(et)


def _sc_gather_u(idxcols, zu, user_bias_flat):
    @functools.partial(
        pl.kernel,
        out_type=(
            jax.ShapeDtypeStruct((NW, BPW, ZW), jnp.float32),
            jax.ShapeDtypeStruct((NW, NCH, CHUNK), jnp.float32),
        ),
        mesh=plsc.VectorSubcoreMesh(**_MESH),
        compiler_params=_PARAMS,
        scratch_types=[
            pltpu.VMEM((NCH, CHUNK), jnp.int32),
            pltpu.VMEM((NCH, CHUNK), jnp.int32),
            pltpu.VMEM((BPW, ZW), jnp.float32),
            pltpu.VMEM((NCH, CHUNK), jnp.float32),
            pltpu.SemaphoreType.DMA,
            pltpu.SemaphoreType.DMA,
        ],
    )
    def ka(idx_h, zu_h, ubias_h, ug_h, ubg_h,
           idxu_v, zru_v, urows_v, ub_v, sem_u, sem_b):
        wid = lax.axis_index("s") * NC + lax.axis_index("c")
        pltpu.sync_copy(idx_h.at[0, wid], idxu_v)
        for j in range(NCH):
            for k in range(CHUNK // LANES):
                sl = pl.ds(k * LANES, LANES)
                ru = idxu_v[j, sl]
                zru_v[j, sl] = ((ru >> 14) << 13) | (ru & (PAIR - 1))
        copies = []
        for j in range(NCH):
            copies.append(pltpu.async_copy(
                zu_h.at[zru_v.at[j]], urows_v.at[pl.ds(j * CHUNK, CHUNK)],
                sem_u))
            copies.append(pltpu.async_copy(
                ubias_h.at[idxu_v.at[j]], ub_v.at[j], sem_b))
        for c in copies:
            c.wait()
        pltpu.sync_copy(urows_v, ug_h.at[wid])
        pltpu.sync_copy(ub_v, ubg_h.at[wid])

    return ka(idxcols, zu, user_bias_flat)


def _sc_dot_v(idxcols, zv, item_bias_flat, ug):
    @functools.partial(
        pl.kernel,
        out_type=(
            jax.ShapeDtypeStruct((NW, LANES), jnp.float32),
            jax.ShapeDtypeStruct((NW, NCH, CHUNK), jnp.float32),
        ),
        mesh=plsc.VectorSubcoreMesh(**_MESH),
        compiler_params=_PARAMS,
        scratch_types=[
            pltpu.VMEM((NCH, CHUNK), jnp.int32),
            pltpu.VMEM((NCH, CHUNK), jnp.int32),
            pltpu.VMEM((NCH, CHUNK), jnp.int32),
            pltpu.VMEM((2, CHUNK, ZW), jnp.float32),
            pltpu.VMEM((2, CHUNK, ZW), jnp.float32),
            pltpu.VMEM((NCH, CHUNK), jnp.float32),
            pltpu.VMEM((LANES,), jnp.float32),
            pltpu.SemaphoreType.DMA,
            pltpu.SemaphoreType.DMA,
            pltpu.SemaphoreType.DMA,
        ],
    )
    def kb(idx_h, zv_h, ibias_h, ug_h, parts_h, ibg_h,
           idxu_v, idxi_v, zri_v, urows_v, vrows_v, ib_v, acc_v,
           sem_u, sem_v, sem_b):
        wid = lax.axis_index("s") * NC + lax.axis_index("c")
        pltpu.sync_copy(idx_h.at[0, wid], idxu_v)
        pltpu.sync_copy(idx_h.at[1, wid], idxi_v)
        for j in range(NCH):
            for k in range(CHUNK // LANES):
                sl = pl.ds(k * LANES, LANES)
                ri = idxi_v[j, sl]
                zri_v[j, sl] = ((ri >> 14) << 13) | (ri & (PAIR - 1))
        bias_copies = []
        for j in range(NCH):
            bias_copies.append(pltpu.async_copy(
                ibias_h.at[idxi_v.at[j]], ib_v.at[j], sem_b))

        def fire(j):
            cu = pltpu.async_copy(
                ug_h.at[wid, pl.ds(j * CHUNK, CHUNK)], urows_v.at[j % 2],
                sem_u)
            cv = pltpu.async_copy(zv_h.at[zri_v.at[j]], vrows_v.at[j % 2],
                                  sem_v)
            return cu, cv

        inflight = fire(0)
        acc = jnp.zeros((LANES,), jnp.float32)
        for j in range(NCH):
            cu, cv = inflight
            if j + 1 < NCH:
                nxt = fire(j + 1)
            cu.wait()
            cv.wait()
            if j + 1 < NCH:
                inflight = nxt
            ub = urows_v.at[j % 2]
            vb = vrows_v.at[j % 2]

            def gbody(g, a, j=j, ub=ub, vb=vb):
                sl = pl.ds(g * LANES, LANES)
                offu16 = ((idxu_v[j, sl] >> 13) & 1) << 6
                offi16 = ((idxi_v[j, sl] >> 13) & 1) << 6
                base = g * LANES
                for ln in range(LANES):
                    su = offu16[ln]
                    si = offi16[ln]
                    row = base + ln
                    p = (ub[row, pl.ds(su, LANES)]
                         * vb[row, pl.ds(si, LANES)])
                    for c in range(1, EMBED // LANES):
                        p = p + (ub[row, pl.ds(su + c * LANES, LANES)]
                                 * vb[row, pl.ds(si + c * LANES, LANES)])
                    a = a + p
                return a

            acc = lax.fori_loop(0, CHUNK // LANES, gbody, acc)
        for c in bias_copies:
            c.wait()
        pltpu.sync_copy(ib_v, ibg_h.at[wid])
        acc_v[...] = acc
        pltpu.sync_copy(acc_v, parts_h.at[wid])

    return kb(idxcols, zv, item_bias_flat, ug)


def _sc_finish(parts, ubg, ibg):
    @functools.partial(
        pl.kernel,
        out_type=jax.ShapeDtypeStruct((NW, BPW), jnp.float32),
        mesh=plsc.VectorSubcoreMesh(**_MESH),
        compiler_params=_PARAMS,
        scratch_types=[
            pltpu.VMEM((NW, LANES), jnp.float32),
            pltpu.VMEM((BPW,), jnp.float32),
            pltpu.VMEM((BPW,), jnp.float32),
            pltpu.VMEM((BPW,), jnp.float32),
        ],
    )
    def fin_kernel(parts_h, ub_h, ib_h, out_h, parts_v, ub_v, ib_v, out_v):
        wid = lax.axis_index("s") * NC + lax.axis_index("c")
        pltpu.sync_copy(parts_h, parts_v)
        pltpu.sync_copy(ub_h.at[wid], ub_v)
        pltpu.sync_copy(ib_h.at[wid], ib_v)
        acc = jnp.zeros((LANES,), jnp.float32)
        for w in range(NW):
            acc = acc + parts_v[w, :]
        s = jnp.sum(acc)
        for g in range(BPW // LANES):
            sl = pl.ds(g * LANES, LANES)
            x = s + ub_v[sl] + ib_v[sl]
            out_v[sl] = 1.0 / (1.0 + jnp.exp(-x))
        pltpu.sync_copy(out_v, out_h.at[wid])

    return fin_kernel(parts, ubg, ibg)


def kernel(inputs, user_embedding, user_bias, item_embedding, item_bias):
    idxcols = inputs.T.reshape(2, NW, NCH, CHUNK)
    zu = _tc_repack(user_embedding.T)
    ug, ubg = _sc_gather_u(idxcols, zu, user_bias.reshape(-1))
    zv = _tc_repack(item_embedding.T)
    parts, ibg = _sc_dot_v(idxcols, zv, item_bias.reshape(-1), ug)
    out = _tc_finish(parts.reshape(NW * LANES // CHUNK, CHUNK),
                     ubg.reshape(CHUNK, CHUNK), ibg.reshape(CHUNK, CHUNK))
    return out.reshape(BATCH, 1)


def _tc_finish(parts2d, ub2d, ib2d):
    def tc_body(parts_ref, ub_ref, ib_ref, out_ref):
        s = jnp.sum(parts_ref[...])
        x = ub_ref[...] + ib_ref[...] + s
        out_ref[...] = 1.0 / (1.0 + jnp.exp(-x))

    return pl.pallas_call(
        tc_body,
        out_shape=jax.ShapeDtypeStruct(ub2d.shape, jnp.float32),
    )(parts2d, ub2d, ib2d)
